# baseline (speedup 1.0000x reference)
import functools

import jax
import jax.numpy as jnp
from jax import lax
from jax.experimental import pallas as pl
from jax.experimental.pallas import tpu as pltpu

B = 256
C = 768
HW = 196
NUM_CLASSES = 1000
E = 16
TOP_K = 2

G = 24
CG = C // G
KC = CG * HW


def _body(x_ref, s_ref, task_ref, wt_ref, brow_ref, bcol_ref,
          logits_ref, weights_ref, sel_ref, mask_ref, acc_ref):
    g = pl.program_id(0)
    acc_ref[g] = jnp.dot(x_ref[...], s_ref[...],
                         preferred_element_type=jnp.float32,
                         precision=lax.Precision.HIGHEST) / jnp.float32(HW)

    @pl.when(g == G - 1)
    def _finish():
        t = task_ref[...]
        t = t - jnp.max(t, axis=-1, keepdims=True)
        te = jnp.exp(t)
        tsm = te / jnp.sum(te, axis=-1, keepdims=True)

        fused = jnp.concatenate(
            [jnp.concatenate([acc_ref[gi] for gi in range(G)], axis=1), tsm],
            axis=1)
        logits = jnp.dot(fused, wt_ref[...],
                         preferred_element_type=jnp.float32) + brow_ref[...]
        logits_ref[...] = logits

        m = jnp.max(logits, axis=-1, keepdims=True)
        pe = jnp.exp(logits - m)
        probs = pe / jnp.sum(pe, axis=-1, keepdims=True)
        lane = lax.broadcasted_iota(jnp.int32, (B, E), 1)
        v1 = jnp.max(probs, axis=-1, keepdims=True)
        i1 = jnp.min(jnp.where(probs == v1, lane, E), axis=-1, keepdims=True)
        p2 = jnp.where(lane == i1, -jnp.inf, probs)
        v2 = jnp.max(p2, axis=-1, keepdims=True)
        i2 = jnp.min(jnp.where(p2 == v2, lane, E), axis=-1, keepdims=True)
        s12 = v1 + v2
        weights_ref[...] = jnp.concatenate([v1 / s12, v2 / s12], axis=1)
        sel_ref[...] = jnp.concatenate([i1, i2], axis=1)

        dn_e = (((0,), (1,)), ((), ()))
        logits_t = lax.dot_general(wt_ref[...], fused, dn_e,
                                   preferred_element_type=jnp.float32)
        logits_t += bcol_ref[...]
        mt = jnp.max(logits_t, axis=0, keepdims=True)
        pet = jnp.exp(logits_t - mt)
        probs_t = pet / jnp.sum(pet, axis=0, keepdims=True)
        erow = lax.broadcasted_iota(jnp.int32, (E, B), 0)
        v1t = jnp.max(probs_t, axis=0, keepdims=True)
        i1t = jnp.min(jnp.where(probs_t == v1t, erow, E), axis=0,
                      keepdims=True)
        p2t = jnp.where(erow == i1t, -jnp.inf, probs_t)
        v2t = jnp.max(p2t, axis=0, keepdims=True)
        i2t = jnp.min(jnp.where(p2t == v2t, erow, E), axis=0, keepdims=True)

        e3 = lax.broadcasted_iota(jnp.int32, (E, TOP_K, B), 0)
        k3 = lax.broadcasted_iota(jnp.int32, (E, TOP_K, B), 1)
        sel3 = jnp.where(k3 == 0, i1t.reshape(1, 1, B), i2t.reshape(1, 1, B))
        mask_ref[...] = (e3 == sel3).astype(jnp.int32)


@functools.partial(jax.jit, static_argnames=("interpret",))
def _run(x2, task_cls, wt, brow, bcol, interpret=False):
    sel = ((jnp.arange(KC, dtype=jnp.int32) // HW)[:, None]
           == jnp.arange(CG, dtype=jnp.int32)[None, :])
    smat = jnp.where(sel, jnp.float32(1.0), jnp.float32(0.0))

    return pl.pallas_call(
        _body,
        grid=(G,),
        in_specs=[
            pl.BlockSpec((B, KC), lambda g: (0, g)),
            pl.BlockSpec((KC, CG), lambda g: (0, 0)),
            pl.BlockSpec((B, NUM_CLASSES), lambda g: (0, 0)),
            pl.BlockSpec((C + NUM_CLASSES, E), lambda g: (0, 0)),
            pl.BlockSpec((1, E), lambda g: (0, 0)),
            pl.BlockSpec((E, 1), lambda g: (0, 0)),
        ],
        out_specs=[
            pl.BlockSpec((B, E), lambda g: (0, 0)),
            pl.BlockSpec((B, TOP_K), lambda g: (0, 0)),
            pl.BlockSpec((B, TOP_K), lambda g: (0, 0)),
            pl.BlockSpec((E, TOP_K, B), lambda g: (0, 0, 0)),
        ],
        out_shape=[
            jax.ShapeDtypeStruct((B, E), jnp.float32),
            jax.ShapeDtypeStruct((B, TOP_K), jnp.float32),
            jax.ShapeDtypeStruct((B, TOP_K), jnp.int32),
            jax.ShapeDtypeStruct((E, TOP_K, B), jnp.int32),
        ],
        scratch_shapes=[pltpu.VMEM((G, B, CG), jnp.float32)],
        interpret=interpret,
    )(x2, smat, task_cls, wt, brow, bcol)


def kernel(hidden_states, task_cls, W, b):
    x2 = hidden_states.reshape(B, C * HW)
    wt = W.T
    brow = b.reshape(1, E)
    bcol = b.reshape(E, 1)
    logits, weights, sel, mask = _run(x2, task_cls, wt, brow, bcol)
    return (logits, weights, sel, mask)


# speedup vs baseline: 1.1498x; 1.1498x over previous
import functools

import jax
import jax.numpy as jnp
from jax import lax
from jax.experimental import pallas as pl
from jax.experimental.pallas import tpu as pltpu

B = 256
C = 768
HW = 196
NUM_CLASSES = 1000
E = 16
TOP_K = 2

G = 24
CG = C // G
KC = CG * HW


def _body(x_ref, task_ref, wt_ref, brow_ref,
          logits_ref, weights_ref, sel_ref, mask_ref, acc_ref):
    g = pl.program_id(0)
    xt = jnp.transpose(x_ref[...], (1, 0))
    csum = jnp.sum(xt.reshape(CG, HW, B), axis=1)
    acc_ref[pl.ds(g * CG, CG), :] = csum * jnp.float32(1.0 / HW)

    @pl.when(g == G - 1)
    def _finish():
        t = task_ref[...]
        t = t - jnp.max(t, axis=-1, keepdims=True)
        te = jnp.exp(t)
        tsm = te / jnp.sum(te, axis=-1, keepdims=True)

        pooled = jnp.transpose(acc_ref[...], (1, 0))
        fused = jnp.concatenate([pooled, tsm], axis=1)
        logits = jnp.dot(fused, wt_ref[...],
                         preferred_element_type=jnp.float32) + brow_ref[...]
        logits_ref[...] = logits

        m = jnp.max(logits, axis=-1, keepdims=True)
        pe = jnp.exp(logits - m)
        probs = pe / jnp.sum(pe, axis=-1, keepdims=True)
        lane = lax.broadcasted_iota(jnp.int32, (B, E), 1)
        v1 = jnp.max(probs, axis=-1, keepdims=True)
        i1 = jnp.min(jnp.where(probs == v1, lane, E), axis=-1, keepdims=True)
        p2 = jnp.where(lane == i1, -jnp.inf, probs)
        v2 = jnp.max(p2, axis=-1, keepdims=True)
        i2 = jnp.min(jnp.where(p2 == v2, lane, E), axis=-1, keepdims=True)
        s12 = v1 + v2
        weights_ref[...] = jnp.concatenate([v1 / s12, v2 / s12], axis=1)
        sel_ref[...] = jnp.concatenate([i1, i2], axis=1)

        selT = jnp.transpose(jnp.concatenate([i1, i2], axis=1), (1, 0))
        e3 = lax.broadcasted_iota(jnp.int32, (E, TOP_K, B), 0)
        mask_ref[...] = (e3 == selT.reshape(1, TOP_K, B)).astype(jnp.int32)


@functools.partial(jax.jit, static_argnames=("interpret",))
def _run(x2, task_cls, wt, brow, interpret=False):
    return pl.pallas_call(
        _body,
        grid=(G,),
        in_specs=[
            pl.BlockSpec((B, KC), lambda g: (0, g)),
            pl.BlockSpec((B, NUM_CLASSES), lambda g: (0, 0)),
            pl.BlockSpec((C + NUM_CLASSES, E), lambda g: (0, 0)),
            pl.BlockSpec((1, E), lambda g: (0, 0)),
        ],
        out_specs=[
            pl.BlockSpec((B, E), lambda g: (0, 0)),
            pl.BlockSpec((B, TOP_K), lambda g: (0, 0)),
            pl.BlockSpec((B, TOP_K), lambda g: (0, 0)),
            pl.BlockSpec((E, TOP_K, B), lambda g: (0, 0, 0)),
        ],
        out_shape=[
            jax.ShapeDtypeStruct((B, E), jnp.float32),
            jax.ShapeDtypeStruct((B, TOP_K), jnp.float32),
            jax.ShapeDtypeStruct((B, TOP_K), jnp.int32),
            jax.ShapeDtypeStruct((E, TOP_K, B), jnp.int32),
        ],
        scratch_shapes=[pltpu.VMEM((C, B), jnp.float32)],
        interpret=interpret,
    )(x2, task_cls, wt, brow)


def kernel(hidden_states, task_cls, W, b):
    x2 = hidden_states.reshape(B, C * HW)
    wt = W.T
    brow = b.reshape(1, E)
    logits, weights, sel, mask = _run(x2, task_cls, wt, brow)
    return (logits, weights, sel, mask)
